# baseline (speedup 1.0000x reference)
import functools

import jax
import jax.numpy as jnp
from jax import lax
from jax.experimental import pallas as pl
from jax.experimental.pallas import tpu as pltpu
from jax.experimental.pallas import tpu_sc as plsc

NC = 2
NS = 16
L = 16
NW = NC * NS

B = 16384
D = 64
BPW = B // NW
GCHUNK = 128
NCHUNK = BPW // GCHUNK


def _body(uid_hbm, iid_hbm, ut_hbm, it_hbm, out_hbm,
          uidx_v, iidx_v, u_rows, i_rows, out_v, sem):
    wid = lax.axis_index("s") * NC + lax.axis_index("c")
    base = wid * BPW

    pltpu.sync_copy(uid_hbm.at[pl.ds(base, BPW)], uidx_v)
    pltpu.sync_copy(iid_hbm.at[pl.ds(base, BPW)], iidx_v)

    handles = []
    for j in range(NCHUNK):
        sl = pl.ds(j * GCHUNK, GCHUNK)
        handles.append(pltpu.async_copy(ut_hbm.at[uidx_v.at[sl]], u_rows.at[sl], sem))
        handles.append(pltpu.async_copy(it_hbm.at[iidx_v.at[sl]], i_rows.at[sl], sem))
    for h in handles:
        h.wait()

    lanes = lax.iota(jnp.int32, L)
    perms = {h: lanes ^ h for h in (8, 4, 2, 1)}
    masks = {h: (lanes & h) != 0 for h in (8, 4, 2, 1)}

    def lperm(v, h):
        return v.at[perms[h]].get(mode="promise_in_bounds", unique_indices=True)

    def group(g, _):
        vs = []
        for rl in range(L):
            r = g * L + rl
            acc = u_rows[r, pl.ds(0, L)] * i_rows[r, pl.ds(0, L)]
            for k in range(1, D // L):
                acc += u_rows[r, pl.ds(k * L, L)] * i_rows[r, pl.ds(k * L, L)]
            vs.append(acc)
        for h in (8, 4, 2, 1):
            half = len(vs) // 2
            vs = [jnp.where(masks[h],
                            vs[p + half] + lperm(vs[p + half], h),
                            vs[p] + lperm(vs[p], h))
                  for p in range(half)]
        out_v[pl.ds(g * L, L)] = vs[0]
        return 0

    lax.fori_loop(0, BPW // L, group, 0)

    pltpu.sync_copy(out_v, out_hbm.at[pl.ds(base, BPW)])


@jax.jit
def _mf_dot(user_id, item_id, user_table, item_table):
    mesh = plsc.VectorSubcoreMesh(core_axis_name="c", subcore_axis_name="s")
    return pl.kernel(
        _body,
        out_type=jax.ShapeDtypeStruct((B,), jnp.float32),
        mesh=mesh,
        compiler_params=pltpu.CompilerParams(use_tc_tiling_on_sc=False),
        scratch_types=[
            pltpu.VMEM((BPW,), jnp.int32),
            pltpu.VMEM((BPW,), jnp.int32),
            pltpu.VMEM((BPW, D), jnp.float32),
            pltpu.VMEM((BPW, D), jnp.float32),
            pltpu.VMEM((BPW,), jnp.float32),
            pltpu.SemaphoreType.DMA,
        ],
    )(user_id, item_id, user_table, item_table)


def kernel(user_id, item_id, user_table, item_table):
    return _mf_dot(user_id, item_id, user_table, item_table)


# speedup vs baseline: 1.5856x; 1.5856x over previous
import jax
import jax.numpy as jnp
from jax import lax
from jax.experimental import pallas as pl
from jax.experimental.pallas import tpu as pltpu
from jax.experimental.pallas import tpu_sc as plsc

NC = 2
NS = 16
L = 16
NW = NC * NS

B = 16384
D = 64
BPW = B // NW
NDRAIN = 4
PASSES = 2
PROWS = BPW // PASSES


def _body(uid_hbm, iid_hbm, ut_hbm, it_hbm, out_hbm,
          uidx_v, iidx_v, u_rows, i_rows, out_v, semu, semi):
    wid = lax.axis_index("s") * NC + lax.axis_index("c")
    base = wid * BPW

    pltpu.sync_copy(uid_hbm.at[pl.ds(base, BPW)], uidx_v)
    pltpu.sync_copy(iid_hbm.at[pl.ds(base, BPW)], iidx_v)

    lanes = lax.iota(jnp.int32, L)
    perms = {h: lanes ^ h for h in (8, 4, 2, 1)}
    masks = {h: (lanes & h) != 0 for h in (8, 4, 2, 1)}

    def lperm(v, h):
        return v.at[perms[h]].get(mode="promise_in_bounds", unique_indices=True)

    for p in range(PASSES):
        pbase = p * PROWS

        def fire(g, _):
            uvec = uidx_v[pl.ds(pbase + g * L, L)]
            ivec = iidx_v[pl.ds(pbase + g * L, L)]
            for rl in range(L):
                r = g * L + rl
                pltpu.async_copy(ut_hbm.at[pl.ds(uvec[rl], 1)],
                                 u_rows.at[pl.ds(r, 1)], semu)
                pltpu.async_copy(it_hbm.at[pl.ds(ivec[rl], 1)],
                                 i_rows.at[pl.ds(r, 1)], semi)
            return 0

        lax.fori_loop(0, PROWS // L, fire, 0)

        chunk = PROWS // NDRAIN
        for j in range(NDRAIN):
            sl = pl.ds(j * chunk, chunk)
            pltpu.make_async_copy(ut_hbm.at[pl.ds(0, chunk)], u_rows.at[sl], semu).wait()
            pltpu.make_async_copy(it_hbm.at[pl.ds(0, chunk)], i_rows.at[sl], semi).wait()

        def group(g, _):
            vs = []
            for rl in range(L):
                r = g * L + rl
                acc = u_rows[r, pl.ds(0, L)] * i_rows[r, pl.ds(0, L)]
                for k in range(1, D // L):
                    acc += u_rows[r, pl.ds(k * L, L)] * i_rows[r, pl.ds(k * L, L)]
                vs.append(acc)
            for h in (8, 4, 2, 1):
                half = len(vs) // 2
                vs = [jnp.where(masks[h],
                                vs[q + half] + lperm(vs[q + half], h),
                                vs[q] + lperm(vs[q], h))
                      for q in range(half)]
            out_v[pl.ds(pbase + g * L, L)] = vs[0]
            return 0

        lax.fori_loop(0, PROWS // L, group, 0)

    pltpu.sync_copy(out_v, out_hbm.at[pl.ds(base, BPW)])


@jax.jit
def _mf_dot(user_id, item_id, user_table, item_table):
    mesh = plsc.VectorSubcoreMesh(core_axis_name="c", subcore_axis_name="s")
    return pl.kernel(
        _body,
        out_type=jax.ShapeDtypeStruct((B,), jnp.float32),
        mesh=mesh,
        scratch_types=[
            pltpu.VMEM((BPW,), jnp.int32),
            pltpu.VMEM((BPW,), jnp.int32),
            pltpu.VMEM((PROWS, D), jnp.float32),
            pltpu.VMEM((PROWS, D), jnp.float32),
            pltpu.VMEM((BPW,), jnp.float32),
            pltpu.SemaphoreType.DMA,
            pltpu.SemaphoreType.DMA,
        ],
    )(user_id, item_id, user_table, item_table)


def kernel(user_id, item_id, user_table, item_table):
    return _mf_dot(user_id, item_id, user_table, item_table)
